# baseline (speedup 1.0000x reference)
import functools

import jax
import jax.numpy as jnp
from jax import lax
from jax.experimental import pallas as pl
from jax.experimental.pallas import tpu as pltpu
from jax.experimental.pallas import tpu_sc as plsc

_PENALTY = 10.0

_L = 16
_NC = 2
_NS = 16
_NW = _NC * _NS
_B, _T, _N = 8, 2048, 128
_BSC = 4
_BTC = _B - _BSC
_TCH = 8
_CT = _T // _TCH
_NTASK = _BSC * _TCH
_NGRP = _N // _L
_UNROLL = 4
_GPL = 4
_BIGT = 1.0e6
_SUB = 2
_SUBT = _CT // _SUB
_KCH = 8
_RCH = _T // _KCH


def _sc_scan_body(x_hbm, out_hbm, buf, stage, sem):
    wid = lax.axis_index("s") * _NC + lax.axis_index("c")
    b = wid // _TCH
    t_base = (wid % _TCH) * _CT

    zeros = jnp.zeros((_L,), jnp.float32)
    ones = jnp.full((_L,), 1.0, jnp.float32)
    neg1 = jnp.full((_L,), -1.0, jnp.float32)
    bigv = jnp.full((_L,), _BIGT, jnp.float32)
    tfv0 = ones * t_base.astype(jnp.float32)

    def step(v, tfv, c, fi, s2, p):
        m = v > 0.0
        mf = jnp.where(m, ones, zeros)
        gap = tfv - p
        c = c + mf
        s2 = s2 + (gap * gap) * mf
        fi = jnp.minimum(fi, jnp.where(m, tfv, bigv))
        p = jnp.where(m, tfv, p)
        return c, fi, s2, p

    def chunk_src(s):
        return x_hbm.at[b, pl.ds(t_base + s * _SUBT, _SUBT), :]

    pltpu.async_copy(chunk_src(0), buf.at[0], sem)

    init = (tfv0,) + (zeros, bigv, zeros, neg1) * _NGRP

    def chunk_body(s, carry):
        slot = lax.rem(s, 2)

        @pl.when(s + 1 < _SUB)
        def _():
            pltpu.async_copy(chunk_src(s + 1), buf.at[lax.rem(s + 1, 2)],
                             sem)

        pltpu.make_async_copy(chunk_src(s), buf.at[slot], sem).wait()

        tfv_in = carry[0]
        out = [None] * len(carry)
        for half in range(2):
            g_base = half * _GPL

            def body(i, hcarry, g_base=g_base, slot=slot):
                tfv = hcarry[0]
                accs = list(hcarry[1:])
                for u in range(_UNROLL):
                    t = i * _UNROLL + u
                    for g in range(_GPL):
                        v = buf[slot, t, pl.ds((g_base + g) * _L, _L)]
                        a = accs[4 * g: 4 * g + 4]
                        accs[4 * g: 4 * g + 4] = list(
                            step(v, tfv, *a))
                    tfv = tfv + ones
                return (tfv,) + tuple(accs)

            lo = 1 + 4 * g_base
            hi = lo + 4 * _GPL
            hinit = (tfv_in,) + tuple(carry[lo:hi])
            hres = lax.fori_loop(0, _SUBT // _UNROLL, body, hinit)
            out[lo:hi] = list(hres[1:])
            out[0] = hres[0]
        return tuple(out)

    res = lax.fori_loop(0, _SUB, chunk_body, init)

    accs = res[1:]
    for g in range(_NGRP):
        c, fi, s2, p = accs[4 * g: 4 * g + 4]
        fic = fi + ones
        s2 = s2 - jnp.where(c > 0.0, fic * fic, zeros)
        col = g * _L
        stage[0, pl.ds(col, _L)] = c
        stage[1, pl.ds(col, _L)] = fi
        stage[2, pl.ds(col, _L)] = p
        stage[3, pl.ds(col, _L)] = s2

    pltpu.sync_copy(stage, out_hbm.at[wid])


def _merge(a, b):
    a_cnt, a_fi, a_la, a_s2 = a
    b_cnt, b_fi, b_la, b_s2 = b
    both = jnp.logical_and(a_cnt > 0.0, b_cnt > 0.0)
    bridge = b_fi - a_la
    s2 = a_s2 + b_s2 + jnp.where(both, bridge * bridge, 0.0)
    return (a_cnt + b_cnt, jnp.minimum(a_fi, b_fi),
            jnp.maximum(a_la, b_la), s2)


def _tc_scan_body(x_ref, out_ref):
    x = x_ref[0]
    shp = (_RCH, _N)
    r_iota = lax.broadcasted_iota(jnp.int32, shp, 0).astype(jnp.float32)
    cnt = jnp.zeros(shp, jnp.float32)
    fi = jnp.full(shp, _BIGT, jnp.float32)
    p = jnp.full(shp, -1.0, jnp.float32)
    s2 = jnp.zeros(shp, jnp.float32)
    for k in range(_KCH):
        v = x[:, k, :]
        tv = r_iota * float(_KCH) + float(k)
        m = v > 0.0
        mf = jnp.where(m, 1.0, 0.0)
        gap = tv - p
        cnt = cnt + mf
        s2 = s2 + (gap * gap) * mf
        fi = jnp.minimum(fi, jnp.where(m, tv, _BIGT))
        p = jnp.where(m, tv, p)
    fic = fi + 1.0
    s2 = s2 - jnp.where(cnt > 0.0, fic * fic, 0.0)

    stats = (cnt, fi, p, s2)
    r = _RCH
    while r > 1:
        split = tuple(s.reshape(r // 2, 2, _N) for s in stats)
        a = tuple(s[:, 0, :] for s in split)
        b = tuple(s[:, 1, :] for s in split)
        stats = _merge(a, b)
        r //= 2

    out_ref[0, 0:1, :] = stats[0]
    out_ref[0, 1:2, :] = stats[1]
    out_ref[0, 2:3, :] = stats[2]
    out_ref[0, 3:4, :] = stats[3]


def _tc_finalize_body(sc_ref, tc_ref, tgt_ref, out_ref):
    acc = tuple(sc_ref[:, 0, k] for k in range(4))
    for c in range(1, _TCH):
        acc = _merge(acc, tuple(sc_ref[:, c, k] for k in range(4)))

    cnt = jnp.concatenate([acc[0], tc_ref[:, 0, :]], axis=0)
    fi = jnp.concatenate([acc[1], tc_ref[:, 1, :]], axis=0)
    la = jnp.concatenate([acc[2], tc_ref[:, 2, :]], axis=0)
    s2 = jnp.concatenate([acc[3], tc_ref[:, 3, :]], axis=0)

    s1 = la - fi
    n_isi = jnp.maximum(cnt - 1.0, 1.0)
    mean = s1 / n_isi
    nv = jnp.maximum(cnt - 1.0, 0.0)
    var_num = jnp.maximum(s2 - 2.0 * mean * s1 + nv * mean * mean, 0.0)
    var = var_num / jnp.maximum(cnt - 2.0, 1.0)
    std = jnp.sqrt(var)
    cv = jnp.where(mean > 0.0, std / jnp.maximum(mean, 1e-30), _PENALTY)
    cvs = jnp.where(cnt >= 3.0, cv, _PENALTY)
    d = cvs - tgt_ref[0][None, :]
    tot = jnp.sum(jnp.sum(d * d, axis=1, keepdims=True), axis=0,
                  keepdims=True)
    out_ref[...] = tot * (1.0 / (_B * _N))


@jax.jit
def kernel(output_spikes, target_cv):
    sc_stats = pl.kernel(
        _sc_scan_body,
        out_type=jax.ShapeDtypeStruct((_NTASK, 4, _N), jnp.float32),
        mesh=plsc.VectorSubcoreMesh(
            core_axis_name="c", subcore_axis_name="s",
            num_cores=_NC, num_subcores=_NS,
        ),
        scratch_types=[
            pltpu.VMEM((2, _SUBT, _N), jnp.float32),
            pltpu.VMEM((4, _N), jnp.float32),
            pltpu.SemaphoreType.DMA,
        ],
    )(output_spikes)
    sc_stats = sc_stats.reshape(_BSC, _TCH, 4, _N)

    x4 = output_spikes.reshape(_B, _RCH, _KCH, _N)
    tc_stats = pl.pallas_call(
        _tc_scan_body,
        grid=(_BTC,),
        in_specs=[pl.BlockSpec((1, _RCH, _KCH, _N),
                               lambda i: (i + _BSC, 0, 0, 0))],
        out_specs=pl.BlockSpec((1, 4, _N), lambda i: (i, 0, 0)),
        out_shape=jax.ShapeDtypeStruct((_BTC, 4, _N), jnp.float32),
    )(x4)

    tgt = target_cv[None, :]
    loss = pl.pallas_call(
        _tc_finalize_body,
        out_shape=jax.ShapeDtypeStruct((1, 1), jnp.float32),
    )(sc_stats, tc_stats, tgt)
    return loss[0, 0]


# speedup vs baseline: 1.3762x; 1.3762x over previous
import functools

import jax
import jax.numpy as jnp
from jax import lax
from jax.experimental import pallas as pl
from jax.experimental.pallas import tpu as pltpu
from jax.experimental.pallas import tpu_sc as plsc

_PENALTY = 10.0

_L = 16
_NC = 2
_NS = 16
_NW = _NC * _NS
_B, _T, _N = 8, 2048, 128
_TC = 4
_CT = _T // _TC
_NGRP = _N // _L
_UNROLL = 8


_BIGT = 1.0e6
_SUB = 4
_SUBT = _CT // _SUB
_GPL = 2


def _sc_scan_body(x_hbm, out_hbm, buf, stage, sem):
    wid = lax.axis_index("s") * _NC + lax.axis_index("c")
    b = wid // _TC
    tc = wid % _TC
    t_base = tc * _CT

    zeros = jnp.zeros((_L,), jnp.float32)
    ones = jnp.full((_L,), 1.0, jnp.float32)
    neg1 = jnp.full((_L,), -1.0, jnp.float32)
    bigv = jnp.full((_L,), _BIGT, jnp.float32)
    t0f = t_base.astype(jnp.float32)
    tfv0 = ones * t0f

    def step(v, tfv, c, fi, s2, p):
        m = v > 0.0
        mf = jnp.where(m, ones, zeros)
        gap = tfv - p
        c = c + mf
        s2 = s2 + (gap * gap) * mf
        fi = jnp.minimum(fi, jnp.where(m, tfv, bigv))
        p = jnp.where(m, tfv, p)
        return c, fi, s2, p

    def chunk_src(s):
        return x_hbm.at[b, pl.ds(t_base + s * _SUBT, _SUBT), :]

    pltpu.async_copy(chunk_src(0), buf.at[0], sem)

    init = (tfv0,) + (zeros, bigv, zeros, neg1) * _NGRP

    def chunk_body(s, carry):
        slot = lax.rem(s, 2)

        @pl.when(s + 1 < _SUB)
        def _():
            pltpu.async_copy(chunk_src(s + 1), buf.at[lax.rem(s + 1, 2)],
                             sem)

        pltpu.make_async_copy(chunk_src(s), buf.at[slot], sem).wait()

        tfv_in = carry[0]
        out = [None] * len(carry)
        for half in range(_NGRP // _GPL):
            g_base = half * _GPL

            def body(i, hcarry, g_base=g_base, slot=slot):
                tfv = hcarry[0]
                accs = list(hcarry[1:])
                for u in range(_UNROLL):
                    t = i * _UNROLL + u
                    for g in range(_GPL):
                        v = buf[slot, t, pl.ds((g_base + g) * _L, _L)]
                        a = accs[4 * g: 4 * g + 4]
                        accs[4 * g: 4 * g + 4] = list(
                            step(v, tfv, *a))
                    tfv = tfv + ones
                return (tfv,) + tuple(accs)

            lo = 1 + 4 * g_base
            hi = lo + 4 * _GPL
            hinit = (tfv_in,) + tuple(carry[lo:hi])
            hres = lax.fori_loop(0, _SUBT // _UNROLL, body, hinit)
            out[lo:hi] = list(hres[1:])
            out[0] = hres[0]
        return tuple(out)

    res = lax.fori_loop(0, _SUB, chunk_body, init)

    accs = res[1:]
    for g in range(_NGRP):
        c, fi, s2, p = accs[4 * g: 4 * g + 4]
        fic = fi + ones
        s2 = s2 - jnp.where(c > 0.0, fic * fic, zeros)
        col = g * _L
        stage[0, pl.ds(col, _L)] = c
        stage[1, pl.ds(col, _L)] = fi
        stage[2, pl.ds(col, _L)] = p
        stage[3, pl.ds(col, _L)] = s2

    pltpu.sync_copy(stage, out_hbm.at[wid])


def _tc_finalize_body(stats_ref, tgt_ref, out_ref):
    cnt = stats_ref[:, 0, 0]
    fi = stats_ref[:, 0, 1]
    la = stats_ref[:, 0, 2]
    s2 = stats_ref[:, 0, 3]
    for c in range(1, _TC):
        cnt_c = stats_ref[:, c, 0]
        fi_c = stats_ref[:, c, 1]
        la_c = stats_ref[:, c, 2]
        s2_c = stats_ref[:, c, 3]
        a_has = cnt > 0.0
        c_has = cnt_c > 0.0
        bridge = fi_c - la
        s2_both = s2 + s2_c + bridge * bridge
        s2 = jnp.where(
            jnp.logical_and(a_has, c_has), s2_both,
            jnp.where(a_has, s2, s2_c))
        fi = jnp.where(a_has, fi, fi_c)
        la = jnp.where(c_has, la_c, la)
        cnt = cnt + cnt_c

    s1 = la - fi
    n_isi = jnp.maximum(cnt - 1.0, 1.0)
    mean = s1 / n_isi
    nv = jnp.maximum(cnt - 1.0, 0.0)
    var_num = jnp.maximum(s2 - 2.0 * mean * s1 + nv * mean * mean, 0.0)
    var = var_num / jnp.maximum(cnt - 2.0, 1.0)
    std = jnp.sqrt(var)
    cv = jnp.where(mean > 0.0, std / jnp.maximum(mean, 1e-30), _PENALTY)
    cvs = jnp.where(cnt >= 3.0, cv, _PENALTY)
    d = cvs - tgt_ref[0][None, :]
    tot = jnp.sum(jnp.sum(d * d, axis=1, keepdims=True), axis=0, keepdims=True)
    out_ref[...] = tot * (1.0 / (_B * _N))


@jax.jit
def kernel(output_spikes, target_cv):
    stats = pl.kernel(
        _sc_scan_body,
        out_type=jax.ShapeDtypeStruct((_NW, 4, _N), jnp.float32),
        mesh=plsc.VectorSubcoreMesh(
            core_axis_name="c", subcore_axis_name="s",
            num_cores=_NC, num_subcores=_NS,
        ),
        scratch_types=[
            pltpu.VMEM((2, _SUBT, _N), jnp.float32),
            pltpu.VMEM((4, _N), jnp.float32),
            pltpu.SemaphoreType.DMA,
        ],
    )(output_spikes)
    stats = stats.reshape(_B, _TC, 4, _N)
    tgt = target_cv[None, :]
    loss = pl.pallas_call(
        _tc_finalize_body,
        out_shape=jax.ShapeDtypeStruct((1, 1), jnp.float32),
    )(stats, tgt)
    return loss[0, 0]


# speedup vs baseline: 1.4314x; 1.0401x over previous
import functools

import jax
import jax.numpy as jnp
from jax import lax
from jax.experimental import pallas as pl
from jax.experimental.pallas import tpu as pltpu
from jax.experimental.pallas import tpu_sc as plsc

_PENALTY = 10.0

_L = 16
_NC = 2
_NS = 16
_NW = _NC * _NS
_B, _T, _N = 8, 2048, 128
_TC = 4
_CT = _T // _TC
_NGRP = _N // _L
_UNROLL = 4


_BIGT = 1.0e6
_SUB = 8
_SUBT = _CT // _SUB
_GPL = 4


def _sc_scan_body(x_hbm, out_hbm, buf, stage, sem):
    wid = lax.axis_index("s") * _NC + lax.axis_index("c")
    b = wid // _TC
    tc = wid % _TC
    t_base = tc * _CT

    zeros = jnp.zeros((_L,), jnp.float32)
    ones = jnp.full((_L,), 1.0, jnp.float32)
    neg1 = jnp.full((_L,), -1.0, jnp.float32)
    bigv = jnp.full((_L,), _BIGT, jnp.float32)
    t0f = t_base.astype(jnp.float32)
    tfv0 = ones * t0f

    def step(v, tfv, c, fi, s2, p):
        m = v > 0.0
        mf = jnp.where(m, ones, zeros)
        gap = tfv - p
        c = c + mf
        s2 = s2 + (gap * gap) * mf
        fi = jnp.minimum(fi, jnp.where(m, tfv, bigv))
        p = jnp.where(m, tfv, p)
        return c, fi, s2, p

    def chunk_src(s):
        return x_hbm.at[b, pl.ds(t_base + s * _SUBT, _SUBT), :]

    pltpu.async_copy(chunk_src(0), buf.at[0], sem)

    init = (tfv0,) + (zeros, bigv, zeros, neg1) * _NGRP

    def chunk_body(s, carry):
        slot = lax.rem(s, 2)

        @pl.when(s + 1 < _SUB)
        def _():
            pltpu.async_copy(chunk_src(s + 1), buf.at[lax.rem(s + 1, 2)],
                             sem)

        pltpu.make_async_copy(chunk_src(s), buf.at[slot], sem).wait()

        tfv_in = carry[0]
        out = [None] * len(carry)
        for half in range(_NGRP // _GPL):
            g_base = half * _GPL

            def body(i, hcarry, g_base=g_base, slot=slot):
                tfv = hcarry[0]
                accs = list(hcarry[1:])
                for u in range(_UNROLL):
                    t = i * _UNROLL + u
                    for g in range(_GPL):
                        v = buf[slot, t, pl.ds((g_base + g) * _L, _L)]
                        a = accs[4 * g: 4 * g + 4]
                        accs[4 * g: 4 * g + 4] = list(
                            step(v, tfv, *a))
                    tfv = tfv + ones
                return (tfv,) + tuple(accs)

            lo = 1 + 4 * g_base
            hi = lo + 4 * _GPL
            hinit = (tfv_in,) + tuple(carry[lo:hi])
            hres = lax.fori_loop(0, _SUBT // _UNROLL, body, hinit)
            out[lo:hi] = list(hres[1:])
            out[0] = hres[0]
        return tuple(out)

    res = lax.fori_loop(0, _SUB, chunk_body, init)

    accs = res[1:]
    for g in range(_NGRP):
        c, fi, s2, p = accs[4 * g: 4 * g + 4]
        fic = fi + ones
        s2 = s2 - jnp.where(c > 0.0, fic * fic, zeros)
        col = g * _L
        stage[0, pl.ds(col, _L)] = c
        stage[1, pl.ds(col, _L)] = fi
        stage[2, pl.ds(col, _L)] = p
        stage[3, pl.ds(col, _L)] = s2

    pltpu.sync_copy(stage, out_hbm.at[wid])


def _tc_finalize_body(stats_ref, tgt_ref, out_ref):
    cnt = stats_ref[:, 0, 0]
    fi = stats_ref[:, 0, 1]
    la = stats_ref[:, 0, 2]
    s2 = stats_ref[:, 0, 3]
    for c in range(1, _TC):
        cnt_c = stats_ref[:, c, 0]
        fi_c = stats_ref[:, c, 1]
        la_c = stats_ref[:, c, 2]
        s2_c = stats_ref[:, c, 3]
        a_has = cnt > 0.0
        c_has = cnt_c > 0.0
        bridge = fi_c - la
        s2_both = s2 + s2_c + bridge * bridge
        s2 = jnp.where(
            jnp.logical_and(a_has, c_has), s2_both,
            jnp.where(a_has, s2, s2_c))
        fi = jnp.where(a_has, fi, fi_c)
        la = jnp.where(c_has, la_c, la)
        cnt = cnt + cnt_c

    s1 = la - fi
    n_isi = jnp.maximum(cnt - 1.0, 1.0)
    mean = s1 / n_isi
    nv = jnp.maximum(cnt - 1.0, 0.0)
    var_num = jnp.maximum(s2 - 2.0 * mean * s1 + nv * mean * mean, 0.0)
    var = var_num / jnp.maximum(cnt - 2.0, 1.0)
    std = jnp.sqrt(var)
    cv = jnp.where(mean > 0.0, std / jnp.maximum(mean, 1e-30), _PENALTY)
    cvs = jnp.where(cnt >= 3.0, cv, _PENALTY)
    d = cvs - tgt_ref[0][None, :]
    tot = jnp.sum(jnp.sum(d * d, axis=1, keepdims=True), axis=0, keepdims=True)
    out_ref[...] = tot * (1.0 / (_B * _N))


@jax.jit
def kernel(output_spikes, target_cv):
    stats = pl.kernel(
        _sc_scan_body,
        out_type=jax.ShapeDtypeStruct((_NW, 4, _N), jnp.float32),
        mesh=plsc.VectorSubcoreMesh(
            core_axis_name="c", subcore_axis_name="s",
            num_cores=_NC, num_subcores=_NS,
        ),
        scratch_types=[
            pltpu.VMEM((2, _SUBT, _N), jnp.float32),
            pltpu.VMEM((4, _N), jnp.float32),
            pltpu.SemaphoreType.DMA,
        ],
    )(output_spikes)
    stats = stats.reshape(_B, _TC, 4, _N)
    tgt = target_cv[None, :]
    loss = pl.pallas_call(
        _tc_finalize_body,
        out_shape=jax.ShapeDtypeStruct((1, 1), jnp.float32),
    )(stats, tgt)
    return loss[0, 0]
